# baseline (speedup 1.0000x reference)
import functools

import jax
import jax.numpy as jnp
from jax import lax
from jax.experimental import pallas as pl
from jax.experimental.pallas import tpu as pltpu
from jax.experimental.pallas import tpu_sc as plsc

VOCAB = 1000
ROW = 1000

NUM_CORES = 2
NUM_SUBCORES = 16
NW = NUM_CORES * NUM_SUBCORES

B_TOTAL = 1024 * 50
B_PER_W = B_TOTAL // NW
GROUP = 32
N_HALF = 2
K_SLOTS = GROUP * N_HALF
N_GROUPS = B_PER_W // GROUP
N_OUTER = N_GROUPS // N_HALF

_mesh = plsc.VectorSubcoreMesh(core_axis_name="c", subcore_axis_name="s")


@functools.partial(
    pl.kernel,
    mesh=_mesh,
    out_type=jax.ShapeDtypeStruct((B_TOTAL, ROW), jnp.float32),
    scratch_types=[
        pltpu.VMEM((B_PER_W,), jnp.int32),
        pltpu.VMEM_SHARED((NUM_SUBCORES, K_SLOTS, ROW), jnp.float32),
        pltpu.SemaphoreType.DMA((N_HALF,)),
        pltpu.SemaphoreType.DMA((N_HALF,)),
    ],
    compiler_params=pltpu.CompilerParams(use_tc_tiling_on_sc=False),
)
def _gather_rows(table_hbm, idx_hbm, out_hbm, idx_v, sp, gsem, wsem):
    cid = lax.axis_index("c")
    sid = lax.axis_index("s")
    wid = sid * NUM_CORES + cid
    base = wid * B_PER_W
    pltpu.sync_copy(idx_hbm.at[pl.ds(base, B_PER_W)], idx_v)

    def issue_group(g, h):
        for v in range(GROUP // 16):
            vec = idx_v[pl.ds(g * GROUP + v * 16, 16)]
            for c in range(16):
                row = vec[c]
                pltpu.async_copy(
                    table_hbm.at[pl.ds(row, 1)],
                    sp.at[sid, pl.ds(h * GROUP + v * 16 + c, 1)],
                    gsem.at[h],
                )

    def drain_group(h):
        for _ in range(GROUP):
            pltpu.make_async_copy(
                table_hbm.at[pl.ds(0, 1)],
                sp.at[sid, pl.ds(h * GROUP, 1)],
                gsem.at[h],
            ).wait()

    def writeback_desc(g, h):
        return pltpu.make_async_copy(
            sp.at[sid, pl.ds(h * GROUP, GROUP)],
            out_hbm.at[pl.ds(base + g * GROUP, GROUP)],
            wsem.at[h],
        )

    issue_group(0, 0)
    issue_group(1, 1)

    def outer(t, _):
        for h in range(N_HALF):
            g = t * N_HALF + h
            drain_group(h)
            writeback_desc(g, h).start()
            nxt = g + N_HALF

            @pl.when(nxt < N_GROUPS)
            def _():
                writeback_desc(nxt - N_HALF, h).wait()
                issue_group(nxt, h)

        return ()

    lax.fori_loop(0, N_OUTER - 1, outer, ())

    for h in range(N_HALF):
        g = (N_OUTER - 1) * N_HALF + h
        drain_group(h)
        writeback_desc(g, h).start()
    for h in range(N_HALF):
        g = (N_OUTER - 1) * N_HALF + h
        writeback_desc(g, h).wait()


def kernel(idx, logits_table):
    flat_idx = idx.reshape(-1).astype(jnp.int32)
    out = _gather_rows(logits_table, flat_idx)
    return out.reshape(idx.shape[0], idx.shape[1], VOCAB)


# speedup vs baseline: 1.0306x; 1.0306x over previous
import functools

import jax
import jax.numpy as jnp
from jax import lax
from jax.experimental import pallas as pl
from jax.experimental.pallas import tpu as pltpu
from jax.experimental.pallas import tpu_sc as plsc

VOCAB = 1000
ROW = 1000

NUM_CORES = 2
NUM_SUBCORES = 16
NW = NUM_CORES * NUM_SUBCORES

B_TOTAL = 1024 * 50
B_PER_W = B_TOTAL // NW
CHUNK = 40
N_BUF = 3
N_CHUNKS = B_PER_W // CHUNK
N_RING = (N_CHUNKS - 1) // N_BUF

_mesh = plsc.VectorSubcoreMesh(core_axis_name="c", subcore_axis_name="s")


@functools.partial(
    pl.kernel,
    mesh=_mesh,
    out_type=jax.ShapeDtypeStruct((B_TOTAL, ROW), jnp.float32),
    scratch_types=[
        pltpu.VMEM((B_PER_W,), jnp.int32),
        pltpu.VMEM((N_BUF, CHUNK, ROW), jnp.float32),
        pltpu.SemaphoreType.DMA((N_BUF,)),
    ],
    compiler_params=pltpu.CompilerParams(use_tc_tiling_on_sc=False),
)
def _gather_rows(table_hbm, idx_hbm, out_hbm, idx_v, rows_v, gsem):
    wid = lax.axis_index("s") * NUM_CORES + lax.axis_index("c")
    base = wid * B_PER_W
    pltpu.sync_copy(idx_hbm.at[pl.ds(base, B_PER_W)], idx_v)

    def gather_desc(i, b):
        return pltpu.make_async_copy(
            table_hbm.at[idx_v.at[pl.ds(i * CHUNK, CHUNK)]],
            rows_v.at[b],
            gsem.at[b],
        )

    def writeback_sync(i, b):
        pltpu.sync_copy(rows_v.at[b], out_hbm.at[pl.ds(base + i * CHUNK, CHUNK)])

    for b in range(N_BUF):
        gather_desc(b, b).start()

    def outer(t, _):
        for b in range(N_BUF):
            i = t * N_BUF + b
            gather_desc(i, b).wait()
            writeback_sync(i, b)
            gather_desc(i + N_BUF, b).start()
        return ()

    lax.fori_loop(0, N_RING - 1, outer, ())

    last = (N_RING - 1) * N_BUF
    gather_desc(last, 0).wait()
    writeback_sync(last, 0)
    gather_desc(N_CHUNKS - 1, 0).start()
    for b in range(1, N_BUF):
        gather_desc(last + b, b).wait()
        writeback_sync(last + b, b)
    gather_desc(N_CHUNKS - 1, 0).wait()
    writeback_sync(N_CHUNKS - 1, 0)


def kernel(idx, logits_table):
    flat_idx = idx.reshape(-1).astype(jnp.int32)
    out = _gather_rows(logits_table, flat_idx)
    return out.reshape(idx.shape[0], idx.shape[1], VOCAB)
